# Initial kernel scaffold; baseline (speedup 1.0000x reference)
import functools

import numpy as np
import jax
import jax.numpy as jnp
from jax import lax
from jax.experimental import pallas as pl
from jax.experimental.pallas import tpu as pltpu
from jax.experimental.pallas import tpu_sc as plsc

_N = 100000
_E = 1600000
_T = 48
_D = 32

_BN = 6250
_LOGT = np.log(np.arange(1, _D + 1, dtype=np.float32)).reshape(_D, 1)
_TS = np.arange(1, _D + 1, dtype=np.float32).reshape(_D, 1)


def _conv_body(x_ref, pt_ref, y_ref):
    p0 = pt_ref[0:1, :]
    p1 = pt_ref[1:2, :]
    s0 = 1.0 / (1.0 + jnp.exp(-p0))
    s1 = 1.0 / (1.0 + jnp.exp(-p1))
    a = 1.0 + (s0 * 0.25 + 0.005) * 10.0
    b = 0.1 + s1 * 1.2
    logk = (a - 1.0) * _LOGT - _TS / b
    k = jnp.exp(logk)
    k = k / (jnp.sum(k, axis=0, keepdims=True) + 1e-8)
    xblk = x_ref[...]
    xp = jnp.concatenate(
        [jnp.zeros((_D - 1, _BN), jnp.float32), xblk], axis=0)
    y = k[0:1, :] * xblk
    for d in range(1, _D):
        y = y + k[d:d + 1, :] * lax.slice_in_dim(xp, _D - 1 - d,
                                                 _D - 1 - d + _T, axis=0)
    y_ref[...] = y


def _conv(x, params_t):
    return pl.pallas_call(
        _conv_body,
        grid=(_N // _BN,),
        in_specs=[
            pl.BlockSpec((_T, _BN), lambda i: (0, i)),
            pl.BlockSpec((2, _BN), lambda i: (0, i)),
        ],
        out_specs=pl.BlockSpec((_T, _BN), lambda i: (0, i)),
        out_shape=jax.ShapeDtypeStruct((_T, _N), jnp.float32),
    )(x, params_t)


_NPART = 25000
_CB = 2000
_EPT = _E // 16
_NCHUNK = _EPT // _CB
_CAP = 2176
_BLK = 128
_ROWS = 2048

_mesh = plsc.VectorSubcoreMesh(core_axis_name="c", subcore_axis_name="s")


@functools.partial(
    pl.kernel,
    out_type=jax.ShapeDtypeStruct((_N, _T), jnp.float32),
    mesh=_mesh,
    scratch_types=[
        pltpu.VMEM_SHARED((_NPART + 16, _T), jnp.float32),
        pltpu.VMEM((_CB,), jnp.int32),
        pltpu.VMEM((_CB,), jnp.int32),
        pltpu.VMEM((_CAP,), jnp.int32),
        pltpu.VMEM((_CAP,), jnp.int32),
        pltpu.VMEM((16, _BLK), jnp.int32),
        pltpu.VMEM((_ROWS, _T), jnp.float32),
        pltpu.SemaphoreType.DMA,
        pltpu.SemaphoreType.DMA,
    ],
)
def _route(yt, src_e, dst_e, out, accum, dstv, srcv, gidx, sidx1, sidx2,
           rows, gsem, ssem):
    c = lax.axis_index("c")
    s = lax.axis_index("s")
    for p in range(2):
        lo = (2 * c + p) * _NPART
        for k in range(2):
            idx = s + 16 * k

            @pl.when(idx < 25)
            def _():
                pltpu.sync_copy(yt.at[pl.ds(lo + idx * 1000, 1000)],
                                accum.at[pl.ds(idx * 1000, 1000)])
        plsc.subcore_barrier()

        trash = _NPART + s

        def chunk_body(ch, _carry):
            base = s * _EPT + ch * _CB
            pltpu.sync_copy(dst_e.at[pl.ds(base, _CB)], dstv)
            pltpu.sync_copy(src_e.at[pl.ds(base, _CB)], srcv)

            def scan_body(i, cnt):
                d = dstv[pl.ds(i * 16, 16)]
                sv = srcv[pl.ds(i * 16, 16)]
                m = jnp.logical_and(d >= lo, d < lo + _NPART)
                plsc.store_compressed(gidx.at[pl.ds(cnt, 16)], sv, mask=m)
                plsc.store_compressed(sidx1.at[pl.ds(cnt, 16)], d - lo,
                                      mask=m)
                return cnt + jnp.sum(jnp.where(m, 1, 0))

            cnt = lax.fori_loop(0, _CB // 16, scan_body, 0)
            zero16 = jnp.zeros((16,), jnp.int32)
            for k in range(_BLK // 16):
                gidx[pl.ds(cnt + k * 16, 16)] = zero16
                sidx1[pl.ds(cnt + k * 16, 16)] = zero16 + trash
            nblk = (cnt + _BLK - 1) // _BLK

            def gfire(j, _):
                pltpu.async_copy(yt.at[gidx.at[pl.ds(j * _BLK, _BLK)]],
                                 rows.at[pl.ds(j * _BLK, _BLK)], gsem)
                return 0

            lax.fori_loop(0, nblk, gfire, 0)

            def gdrain(j, _):
                pltpu.make_async_copy(
                    yt.at[gidx.at[pl.ds(j * _BLK, _BLK)]],
                    rows.at[pl.ds(j * _BLK, _BLK)], gsem).wait()
                return 0

            lax.fori_loop(0, nblk, gdrain, 0)

            def sfire(j, _):
                pltpu.sync_copy(sidx1.at[pl.ds(j * _BLK, _BLK)], sidx2.at[j])
                pltpu.async_copy(rows.at[pl.ds(j * _BLK, _BLK)],
                                 accum.at[sidx2.at[j]], ssem, add=True)
                return 0

            lax.fori_loop(0, nblk, sfire, 0)

            def sdrain(j, _):
                pltpu.make_async_copy(rows.at[pl.ds(j * _BLK, _BLK)],
                                      accum.at[sidx2.at[j]], ssem,
                                      add=True).wait()
                return 0

            lax.fori_loop(0, nblk, sdrain, 0)
            return 0

        lax.fori_loop(0, _NCHUNK, chunk_body, 0)
        plsc.subcore_barrier()
        for k in range(2):
            idx = s + 16 * k

            @pl.when(idx < 25)
            def _():
                pltpu.sync_copy(accum.at[pl.ds(idx * 1000, 1000)],
                                out.at[pl.ds(lo + idx * 1000, 1000)])
        plsc.subcore_barrier()


def kernel(x, edge_index, params):
    y = _conv(x, params.T)
    yt = y.T
    routed = _route(yt, edge_index[0], edge_index[1])
    return routed.T


# speedup vs baseline: 1.5420x; 1.5420x over previous
import functools

import numpy as np
import jax
import jax.numpy as jnp
from jax import lax
from jax.experimental import pallas as pl
from jax.experimental.pallas import tpu as pltpu
from jax.experimental.pallas import tpu_sc as plsc

_N = 100000
_E = 1600000
_T = 48
_D = 32

_NPAD = 102400
_BN = 6400


def _conv_body(x_ref, pt_ref, y_ref):
    ts = lax.broadcasted_iota(jnp.int32, (_D, 1), 0).astype(jnp.float32) + 1.0
    logt = jnp.log(ts)
    p0 = pt_ref[0:1, :]
    p1 = pt_ref[1:2, :]
    s0 = 1.0 / (1.0 + jnp.exp(-p0))
    s1 = 1.0 / (1.0 + jnp.exp(-p1))
    a = 1.0 + (s0 * 0.25 + 0.005) * 10.0
    b = 0.1 + s1 * 1.2
    logk = (a - 1.0) * logt - ts / b
    k = jnp.exp(logk)
    k = k / (jnp.sum(k, axis=0, keepdims=True) + 1e-8)
    xblk = x_ref[...]
    xp = jnp.concatenate(
        [jnp.zeros((_D - 1, _BN), jnp.float32), xblk], axis=0)
    y = k[0:1, :] * xblk
    for d in range(1, _D):
        y = y + k[d:d + 1, :] * lax.slice_in_dim(xp, _D - 1 - d,
                                                 _D - 1 - d + _T, axis=0)
    y_ref[...] = y


def _conv(x, params_t):
    return pl.pallas_call(
        _conv_body,
        grid=(_NPAD // _BN,),
        in_specs=[
            pl.BlockSpec((_T, _BN), lambda i: (0, i)),
            pl.BlockSpec((2, _BN), lambda i: (0, i)),
        ],
        out_specs=pl.BlockSpec((_T, _BN), lambda i: (0, i)),
        out_shape=jax.ShapeDtypeStruct((_T, _NPAD), jnp.float32),
    )(x, params_t)


_NPART = 25000
_CB = 800
_EPT = _E // 16
_NCHUNK = _EPT // _CB
_CAP = 944
_BLK = 128
_ROWS = 896
_NBLK = _ROWS // _BLK

_mesh = plsc.VectorSubcoreMesh(core_axis_name="c", subcore_axis_name="s")


@functools.partial(
    pl.kernel,
    out_type=jax.ShapeDtypeStruct((_N, _T), jnp.float32),
    mesh=_mesh,
    compiler_params=pltpu.CompilerParams(needs_layout_passes=False,
                                         use_tc_tiling_on_sc=False),
    scratch_types=[
        pltpu.VMEM_SHARED((_NPART + 16, _T), jnp.float32),
        pltpu.VMEM((_CB,), jnp.int32),
        pltpu.VMEM((_CB,), jnp.int32),
        pltpu.VMEM((_CAP,), jnp.int32),
        pltpu.VMEM((_NBLK + 1, _BLK), jnp.int32),
        pltpu.VMEM((_ROWS, _T), jnp.float32),
        pltpu.SemaphoreType.DMA,
        pltpu.SemaphoreType.DMA,
    ],
)
def _route(yt, src_e, dst_e, out, accum, dstv, srcv, gidx, sidx2, rows,
           gsem, ssem):
    c = lax.axis_index("c")
    s = lax.axis_index("s")
    for p in range(2):
        lo = (2 * c + p) * _NPART
        for k in range(2):
            idx = s + 16 * k

            @pl.when(idx < 25)
            def _():
                pltpu.sync_copy(yt.at[pl.ds(lo + idx * 1000, 1000)],
                                accum.at[pl.ds(idx * 1000, 1000)])
        plsc.subcore_barrier()

        trash = _NPART + s

        def chunk_body(ch, _carry):
            base = s * _EPT + ch * _CB
            pltpu.sync_copy(dst_e.at[pl.ds(base, _CB)], dstv)
            pltpu.sync_copy(src_e.at[pl.ds(base, _CB)], srcv)

            def scan_body(i, cnt):
                d = dstv[pl.ds(i * 16, 16)]
                sv = srcv[pl.ds(i * 16, 16)]
                m = jnp.logical_and(d >= lo, d < lo + _NPART)
                mi = jnp.where(m, 1, 0)
                pos = cnt + plsc.cumsum(mi) - 1
                plsc.store_scatter(gidx, [pos], sv, mask=m)
                plsc.store_scatter(sidx2, [pos >> 7, pos & 127], d - lo,
                                   mask=m)
                return cnt + jnp.sum(mi)

            cnt = lax.fori_loop(0, _CB // 16, scan_body, 0)
            zero16 = jnp.zeros((16,), jnp.int32)
            for k in range(_BLK // 16):
                ppos = cnt + k * 16 + lax.iota(jnp.int32, 16)
                gidx[pl.ds(cnt + k * 16, 16)] = zero16
                plsc.store_scatter(sidx2, [ppos >> 7, ppos & 127],
                                   zero16 + trash)
            nblk = (cnt + _BLK - 1) // _BLK

            def gfire(j, _):
                pltpu.async_copy(yt.at[gidx.at[pl.ds(j * _BLK, _BLK)]],
                                 rows.at[pl.ds(j * _BLK, _BLK)], gsem)
                return 0

            lax.fori_loop(0, nblk, gfire, 0)

            def gdrain(j, _):
                pltpu.make_async_copy(
                    yt.at[gidx.at[pl.ds(j * _BLK, _BLK)]],
                    rows.at[pl.ds(j * _BLK, _BLK)], gsem).wait()
                return 0

            lax.fori_loop(0, nblk, gdrain, 0)

            def sfire(j, _):
                pltpu.async_copy(rows.at[pl.ds(j * _BLK, _BLK)],
                                 accum.at[sidx2.at[j]], ssem, add=True)
                return 0

            lax.fori_loop(0, nblk, sfire, 0)

            def sdrain(j, _):
                pltpu.make_async_copy(rows.at[pl.ds(j * _BLK, _BLK)],
                                      accum.at[sidx2.at[j]], ssem).wait()
                return 0

            lax.fori_loop(0, nblk, sdrain, 0)
            return 0

        lax.fori_loop(0, _NCHUNK, chunk_body, 0)
        plsc.subcore_barrier()
        for k in range(2):
            idx = s + 16 * k

            @pl.when(idx < 25)
            def _():
                pltpu.sync_copy(accum.at[pl.ds(idx * 1000, 1000)],
                                out.at[pl.ds(lo + idx * 1000, 1000)])
        plsc.subcore_barrier()


def kernel(x, edge_index, params):
    xpad = jnp.pad(x, ((0, 0), (0, _NPAD - _N)))
    ppad = jnp.pad(params.T, ((0, 0), (0, _NPAD - _N)))
    y = _conv(xpad, ppad)
    yt = y[:, :_N].T
    routed = _route(yt, edge_index[0], edge_index[1])
    return routed.T
